# baseline (speedup 1.0000x reference)
import functools

import jax
import jax.numpy as jnp
from jax.experimental import pallas as pl
from jax.experimental.pallas import tpu as pltpu


BLK = 400


def _elu(v):
    return jnp.where(v > 0, v, jnp.exp(jnp.minimum(v, 0.0)) - 1.0)


def _lin1_body(x_ref, w_ref, b_ref, o_ref):
    o_ref[...] = (
        jnp.dot(x_ref[...], w_ref[...], preferred_element_type=jnp.float32)
        + b_ref[...]
    )


def _lin1(x, W, b):
    n, d = x.shape
    return pl.pallas_call(
        _lin1_body,
        grid=(n // BLK,),
        in_specs=[
            pl.BlockSpec((BLK, d), lambda i: (i, 0)),
            pl.BlockSpec((d, W.shape[1]), lambda i: (0, 0)),
            pl.BlockSpec((1, W.shape[1]), lambda i: (0, 0)),
        ],
        out_specs=pl.BlockSpec((BLK, W.shape[1]), lambda i: (i, 0)),
        out_shape=jax.ShapeDtypeStruct((n, W.shape[1]), jnp.float32),
    )(x, W, b.reshape(1, -1))


def _layer_body(h_ref, S_ref, M_ref, G_ref, invc_ref, naW_ref, nab_ref,
                naw_ref, l2W_ref, l2b_ref, o_ref):
    h = h_ref[...]
    acc = None
    for br in range(3):
        S = S_ref[br]
        aggs = (S * invc_ref[br], M_ref[br], G_ref[br], h + S)
        mixed = None
        for p in range(4):
            t = _elu(
                jnp.dot(aggs[p], naW_ref[br, p],
                        preferred_element_type=jnp.float32)
                + nab_ref[br, p]
            )
            t = naw_ref[0, p] * t
            mixed = t if mixed is None else mixed + t
        part = jnp.dot(mixed, l2W_ref[br], preferred_element_type=jnp.float32)
        acc = part if acc is None else acc + part
    o_ref[...] = jnp.maximum(acc + l2b_ref[...], 0.0)


def _layer_dense(h, S, M, G, invc, naW, nab, naw, l2W, l2b):
    n, d = h.shape
    return pl.pallas_call(
        _layer_body,
        grid=(n // BLK,),
        in_specs=[
            pl.BlockSpec((BLK, d), lambda i: (i, 0)),
            pl.BlockSpec((3, BLK, d), lambda i: (0, i, 0)),
            pl.BlockSpec((3, BLK, d), lambda i: (0, i, 0)),
            pl.BlockSpec((3, BLK, d), lambda i: (0, i, 0)),
            pl.BlockSpec((3, BLK, 1), lambda i: (0, i, 0)),
            pl.BlockSpec((3, 4, d, d), lambda i: (0, 0, 0, 0)),
            pl.BlockSpec((3, 4, 1, d), lambda i: (0, 0, 0, 0)),
            pl.BlockSpec((1, 4), lambda i: (0, 0)),
            pl.BlockSpec((3, d, d), lambda i: (0, 0, 0)),
            pl.BlockSpec((1, d), lambda i: (0, 0)),
        ],
        out_specs=pl.BlockSpec((BLK, d), lambda i: (i, 0)),
        out_shape=jax.ShapeDtypeStruct((n, d), jnp.float32),
    )(h, S, M, G, invc, naW, nab.reshape(3, 4, 1, d), naw.reshape(1, 4),
      l2W, l2b.reshape(1, d))


def _head_body(hs_ref, scw_ref, law_ref, laW_ref, lab_ref,
               c1W_ref, c1b_ref, c2W_ref, c2b_ref, o_ref):
    jk = [scw_ref[0, i] * hs_ref[i] for i in range(3)]
    l_max = jnp.maximum(jnp.maximum(jk[0], jk[1]), jk[2])
    l_mean = (jk[0] + jk[1] + jk[2]) * (1.0 / 3.0)
    l_cat = None
    for i in range(3):
        t = jnp.dot(jk[i], laW_ref[i], preferred_element_type=jnp.float32)
        l_cat = t if l_cat is None else l_cat + t
    l_cat = l_cat + lab_ref[...]
    merge = (law_ref[0, 0] * jnp.maximum(l_max, 0.0)
             + law_ref[0, 1] * jnp.maximum(l_mean, 0.0)
             + law_ref[0, 2] * jnp.maximum(l_cat, 0.0))
    merge = jnp.maximum(merge, 0.0)
    hid = jnp.maximum(
        jnp.dot(merge, c1W_ref[...], preferred_element_type=jnp.float32)
        + c1b_ref[...], 0.0)
    o_ref[...] = (
        jnp.dot(hid, c2W_ref[...], preferred_element_type=jnp.float32)
        + c2b_ref[...])


def _head(hs, scw, law, laW, lab, c1W, c1b, c2W, c2b):
    _, n, d = hs.shape
    od = c2W.shape[1]
    return pl.pallas_call(
        _head_body,
        grid=(n // BLK,),
        in_specs=[
            pl.BlockSpec((3, BLK, d), lambda i: (0, i, 0)),
            pl.BlockSpec((1, 3), lambda i: (0, 0)),
            pl.BlockSpec((1, 3), lambda i: (0, 0)),
            pl.BlockSpec((3, d, d), lambda i: (0, 0, 0)),
            pl.BlockSpec((1, d), lambda i: (0, 0)),
            pl.BlockSpec((d, d), lambda i: (0, 0)),
            pl.BlockSpec((1, d), lambda i: (0, 0)),
            pl.BlockSpec((d, od), lambda i: (0, 0)),
            pl.BlockSpec((1, od), lambda i: (0, 0)),
        ],
        out_specs=pl.BlockSpec((BLK, od), lambda i: (i, 0)),
        out_shape=jax.ShapeDtypeStruct((n, od), jnp.float32),
    )(hs, scw.reshape(1, 3), law.reshape(1, 3), laW, lab.reshape(1, d),
      c1W, c1b.reshape(1, d), c2W, c2b.reshape(1, od))


def kernel(x, edge_index, na_alphas, sc_alphas, la_alphas, lin1_W, lin1_b,
           lin2_W, lin2_b, na_W, na_b, la_W, la_b, cls_W1, cls_b1,
           cls_W2, cls_b2):
    n, d = x.shape
    L = na_W.shape[0]
    na_w = jax.nn.softmax(na_alphas, axis=-1)
    sc_w = jax.nn.softmax(sc_alphas, axis=-1)
    la_w = jax.nn.softmax(la_alphas, axis=-1)

    srcs = [edge_index[br, 0] for br in range(3)]
    dsts = [edge_index[br, 1] for br in range(3)]
    cnts = [
        jax.ops.segment_sum(jnp.ones((srcs[br].shape[0],), jnp.float32),
                            dsts[br], num_segments=n)
        for br in range(3)
    ]
    invc = jnp.stack([1.0 / jnp.maximum(c, 1.0) for c in cnts])[..., None]
    degs = [jnp.maximum(c, 1.0) for c in cnts]
    norms = [
        jax.lax.rsqrt(degs[br][srcs[br]] * degs[br][dsts[br]])
        for br in range(3)
    ]

    h = _lin1(x, lin1_W, lin1_b)
    l2W = lin2_W.reshape(3, d, d)
    laW = la_W.reshape(L, d, d)

    hs = []
    for i in range(L):
        Ss, Ms, Gs = [], [], []
        for br in range(3):
            msg = jnp.take(h, srcs[br], axis=0)
            S = jax.ops.segment_sum(msg, dsts[br], num_segments=n)
            M = jax.ops.segment_max(msg, dsts[br], num_segments=n)
            M = jnp.where(jnp.isfinite(M), M, 0.0)
            G = jax.ops.segment_sum(msg * norms[br][:, None], dsts[br],
                                    num_segments=n)
            Ss.append(S)
            Ms.append(M)
            Gs.append(G)
        h = _layer_dense(h, jnp.stack(Ss), jnp.stack(Ms), jnp.stack(Gs),
                         invc, na_W[i], na_b[i], na_w[i], l2W, lin2_b)
        hs.append(h)

    return _head(jnp.stack(hs), sc_w[:, 1], la_w[0], laW, la_b,
                 cls_W1, cls_b1, cls_W2, cls_b2)


# speedup vs baseline: 2.3389x; 2.3389x over previous
import functools

import jax
import jax.numpy as jnp
from jax import lax
from jax.experimental import pallas as pl
from jax.experimental.pallas import tpu as pltpu
from jax.experimental.pallas import tpu_sc as plsc


BLK = 400
NW = 32
RPT = 2
NT = NW * RPT
NPT = 160
NPAD = NT * NPT
CHUNK = 128
NEG = -1.0e30


def _elu(v):
    return jnp.where(v > 0, v, jnp.exp(jnp.minimum(v, 0.0)) - 1.0)


def _sc_body(EPAD, htab, dstp, srcp, rsep, meta, S, M, G,
             outS, outM, outG, stage, srcb, dstb, rseb, metab, sem0, sem1):
    cid = lax.axis_index("c")
    sid = lax.axis_index("s")
    wid = sid * 2 + cid
    sems = (sem0, sem1)

    def pass_body(it, _):
        br = it >> 1
        rr = it & 1
        if True:
            tile = wid * RPT + rr
            moff = pl.multiple_of((br * NT + tile) * 16, 16)
            pltpu.sync_copy(meta.at[pl.ds(moff, 16)], metab)
            mv = metab[...]
            e0 = pl.multiple_of(mv[0] + br * EPAD, 8)
            nc = mv[1]
            vlo = tile * NPT

            def zbody(r, _):
                z = jnp.zeros((16,), jnp.float32)
                neg = jnp.full((16,), NEG, jnp.float32)
                for j in range(8):
                    sl = pl.ds(j * 16, 16)
                    outS[r, sl] = z
                    outG[r, sl] = z
                    outM[r, sl] = neg
                return 0

            lax.fori_loop(0, NPT + 8, zbody, 0)

            def load_idx(chunk, b):
                off = pl.multiple_of(e0 + chunk * CHUNK, 8)
                pltpu.sync_copy(srcp.at[pl.ds(off, CHUNK)], srcb.at[b])
                pltpu.sync_copy(dstp.at[pl.ds(off, CHUNK)], dstb.at[b])
                pltpu.sync_copy(rsep.at[pl.ds(off, CHUNK)], rseb.at[b])

            def gather(b):
                return pltpu.make_async_copy(
                    htab.at[srcb.at[b]], stage.at[b], sems[b])

            load_idx(0, 0)
            gather(0).start()

            def outer(c2, _):
                for b in range(2):
                    chunk = c2 * 2 + b

                    @pl.when(chunk < nc)
                    def _():
                        @pl.when(chunk + 1 < nc)
                        def _():
                            load_idx(chunk + 1, 1 - b)
                            gather(1 - b).start()

                        gather(b).wait()

                        def gbody(g, _):
                            base = g * 16
                            dv = dstb[b, pl.ds(base, 16)]
                            wv = rseb[b, pl.ds(base, 16)]
                            okv = (dv >= vlo) & (dv < vlo + NPT)
                            slotv = jnp.where(okv, dv - vlo, NPT)
                            for k in range(16):
                                slot = slotv[k]
                                w = wv[k]
                                e = base + k
                                for j in range(8):
                                    sl = pl.ds(j * 16, 16)
                                    r = stage[b, e, sl]
                                    plsc.addupdate(outS.at[slot, sl], r)
                                    plsc.addupdate(outG.at[slot, sl], w * r)
                                    outM[slot, sl] = jnp.maximum(
                                        outM[slot, sl], r)
                            return 0

                        lax.fori_loop(0, CHUNK // 16, gbody, 0)
                return 0

            lax.fori_loop(0, (nc + 1) // 2, outer, 0)

            vo = pl.multiple_of(vlo, 8)
            pltpu.sync_copy(outS.at[pl.ds(0, NPT)], S.at[br, pl.ds(vo, NPT)])
            pltpu.sync_copy(outM.at[pl.ds(0, NPT)], M.at[br, pl.ds(vo, NPT)])
            pltpu.sync_copy(outG.at[pl.ds(0, NPT)], G.at[br, pl.ds(vo, NPT)])
        return 0

    lax.fori_loop(0, 3 * RPT, pass_body, 0)


def _sc_reduce(h, dstp, srcp, rsep, meta):
    out = jax.ShapeDtypeStruct((3, NPAD, 128), jnp.float32)
    k = pl.kernel(
        functools.partial(_sc_body, dstp.shape[0] // 3),
        out_type=[out, out, out],
        mesh=plsc.VectorSubcoreMesh(core_axis_name="c", subcore_axis_name="s"),
        scratch_types=[
            pltpu.VMEM((NPT + 8, 128), jnp.float32),
            pltpu.VMEM((NPT + 8, 128), jnp.float32),
            pltpu.VMEM((NPT + 8, 128), jnp.float32),
            pltpu.VMEM((2, CHUNK, 128), jnp.float32),
            pltpu.VMEM((2, CHUNK), jnp.int32),
            pltpu.VMEM((2, CHUNK), jnp.int32),
            pltpu.VMEM((2, CHUNK), jnp.float32),
            pltpu.VMEM((16,), jnp.int32),
            pltpu.SemaphoreType.DMA,
            pltpu.SemaphoreType.DMA,
        ],
    )
    return k(h, dstp, srcp, rsep, meta)


def _lin1_body(x_ref, w_ref, b_ref, o_ref):
    o_ref[...] = (
        jnp.dot(x_ref[...], w_ref[...], preferred_element_type=jnp.float32)
        + b_ref[...])


def _lin1(x, W, b):
    n, d = x.shape
    return pl.pallas_call(
        _lin1_body,
        grid=(n // BLK,),
        in_specs=[
            pl.BlockSpec((BLK, d), lambda i: (i, 0)),
            pl.BlockSpec((d, d), lambda i: (0, 0)),
            pl.BlockSpec((1, d), lambda i: (0, 0)),
        ],
        out_specs=pl.BlockSpec((BLK, d), lambda i: (i, 0)),
        out_shape=jax.ShapeDtypeStruct((n, d), jnp.float32),
    )(x, W, b.reshape(1, -1))


def _layer_body(h_ref, S_ref, M_ref, G_ref, invc_ref, rd_ref, naW_ref,
                nab_ref, naw_ref, l2W_ref, l2b_ref, o_ref):
    h = h_ref[...]
    acc = None
    for br in range(3):
        S = S_ref[br]
        Mr = M_ref[br]
        Mr = jnp.where(Mr > -1.0e29, Mr, 0.0)
        Gr = G_ref[br]
        aggs = (S * invc_ref[br], Mr, Gr * rd_ref[br], h + S)
        mixed = None
        for p in range(4):
            t = _elu(
                jnp.dot(aggs[p], naW_ref[br, p],
                        preferred_element_type=jnp.float32)
                + nab_ref[br, p]
            )
            t = naw_ref[0, p] * t
            mixed = t if mixed is None else mixed + t
        part = jnp.dot(mixed, l2W_ref[br], preferred_element_type=jnp.float32)
        acc = part if acc is None else acc + part
    o_ref[...] = jnp.maximum(acc + l2b_ref[...], 0.0)


def _layer_dense(h, S, M, G, invc, rd, naW, nab, naw, l2W, l2b):
    n, d = h.shape
    return pl.pallas_call(
        _layer_body,
        grid=(n // BLK,),
        in_specs=[
            pl.BlockSpec((BLK, d), lambda i: (i, 0)),
            pl.BlockSpec((3, BLK, d), lambda i: (0, i, 0)),
            pl.BlockSpec((3, BLK, d), lambda i: (0, i, 0)),
            pl.BlockSpec((3, BLK, d), lambda i: (0, i, 0)),
            pl.BlockSpec((3, BLK, 1), lambda i: (0, i, 0)),
            pl.BlockSpec((3, BLK, 1), lambda i: (0, i, 0)),
            pl.BlockSpec((3, 4, d, d), lambda i: (0, 0, 0, 0)),
            pl.BlockSpec((3, 4, 1, d), lambda i: (0, 0, 0, 0)),
            pl.BlockSpec((1, 4), lambda i: (0, 0)),
            pl.BlockSpec((3, d, d), lambda i: (0, 0, 0)),
            pl.BlockSpec((1, d), lambda i: (0, 0)),
        ],
        out_specs=pl.BlockSpec((BLK, d), lambda i: (i, 0)),
        out_shape=jax.ShapeDtypeStruct((n, d), jnp.float32),
    )(h, S, M, G, invc, rd, naW, nab.reshape(3, 4, 1, d), naw.reshape(1, 4),
      l2W, l2b.reshape(1, d))


def _head_body(hs_ref, scw_ref, law_ref, laW_ref, lab_ref,
               c1W_ref, c1b_ref, c2W_ref, c2b_ref, o_ref):
    jk = [scw_ref[0, i] * hs_ref[i] for i in range(3)]
    l_max = jnp.maximum(jnp.maximum(jk[0], jk[1]), jk[2])
    l_mean = (jk[0] + jk[1] + jk[2]) * (1.0 / 3.0)
    l_cat = None
    for i in range(3):
        t = jnp.dot(jk[i], laW_ref[i], preferred_element_type=jnp.float32)
        l_cat = t if l_cat is None else l_cat + t
    l_cat = l_cat + lab_ref[...]
    merge = (law_ref[0, 0] * jnp.maximum(l_max, 0.0)
             + law_ref[0, 1] * jnp.maximum(l_mean, 0.0)
             + law_ref[0, 2] * jnp.maximum(l_cat, 0.0))
    merge = jnp.maximum(merge, 0.0)
    hid = jnp.maximum(
        jnp.dot(merge, c1W_ref[...], preferred_element_type=jnp.float32)
        + c1b_ref[...], 0.0)
    o_ref[...] = (
        jnp.dot(hid, c2W_ref[...], preferred_element_type=jnp.float32)
        + c2b_ref[...])


def _head(hs, scw, law, laW, lab, c1W, c1b, c2W, c2b):
    _, n, d = hs.shape
    od = c2W.shape[1]
    return pl.pallas_call(
        _head_body,
        grid=(n // BLK,),
        in_specs=[
            pl.BlockSpec((3, BLK, d), lambda i: (0, i, 0)),
            pl.BlockSpec((1, 3), lambda i: (0, 0)),
            pl.BlockSpec((1, 3), lambda i: (0, 0)),
            pl.BlockSpec((3, d, d), lambda i: (0, 0, 0)),
            pl.BlockSpec((1, d), lambda i: (0, 0)),
            pl.BlockSpec((d, d), lambda i: (0, 0)),
            pl.BlockSpec((1, d), lambda i: (0, 0)),
            pl.BlockSpec((d, od), lambda i: (0, 0)),
            pl.BlockSpec((1, od), lambda i: (0, 0)),
        ],
        out_specs=pl.BlockSpec((BLK, od), lambda i: (i, 0)),
        out_shape=jax.ShapeDtypeStruct((n, od), jnp.float32),
    )(hs, scw.reshape(1, 3), law.reshape(1, 3), laW, lab.reshape(1, d),
      c1W, c1b.reshape(1, d), c2W, c2b.reshape(1, od))


def kernel(x, edge_index, na_alphas, sc_alphas, la_alphas, lin1_W, lin1_b,
           lin2_W, lin2_b, na_W, na_b, la_W, la_b, cls_W1, cls_b1,
           cls_W2, cls_b2):
    n, d = x.shape
    L = na_W.shape[0]
    E = edge_index.shape[2]
    EP = E + 2 * CHUNK
    na_w = jax.nn.softmax(na_alphas, axis=-1)
    sc_w = jax.nn.softmax(sc_alphas, axis=-1)
    la_w = jax.nn.softmax(la_alphas, axis=-1)

    dstps, srcps, rseps, metas, invcs, rds = [], [], [], [], [], []
    vb = jnp.minimum(jnp.arange(NT + 1, dtype=jnp.int32) * NPT, n)
    for br in range(3):
        src = edge_index[br, 0]
        dst = edge_index[br, 1]
        dst_s, src_s = lax.sort((dst, src), num_keys=1)
        ptr = jnp.searchsorted(dst_s, jnp.arange(n + 1, dtype=jnp.int32),
                               side="left").astype(jnp.int32)
        cnt = (ptr[1:] - ptr[:-1]).astype(jnp.float32)
        deg = jnp.maximum(cnt, 1.0)
        rs = lax.rsqrt(deg)
        invcs.append(1.0 / deg)
        rds.append(rs)
        rse = rs[src_s]
        e_b = ptr[vb]
        e_lo = e_b[:NT] & ~jnp.int32(7)
        nc = jnp.maximum((e_b[1:] - e_lo + (CHUNK - 1)) // CHUNK, 1)
        metas.append(jnp.stack(
            [e_lo, nc, jnp.zeros_like(e_lo), jnp.zeros_like(e_lo)], axis=1))
        dstps.append(jnp.concatenate(
            [dst_s, jnp.full((EP - E,), NPAD, jnp.int32)]))
        srcps.append(jnp.concatenate([src_s, jnp.zeros((EP - E,), jnp.int32)]))
        rseps.append(jnp.concatenate([rse, jnp.zeros((EP - E,), jnp.float32)]))
    dstp = jnp.concatenate(dstps)
    srcp = jnp.concatenate(srcps)
    rsep = jnp.concatenate(rseps)
    meta = jnp.pad(jnp.stack(metas), ((0, 0), (0, 0), (0, 12))).reshape(-1)
    invc = jnp.stack(invcs)[..., None]
    rd = jnp.stack(rds)[..., None]

    h = _lin1(x, lin1_W, lin1_b)
    l2W = lin2_W.reshape(3, d, d)
    laW = la_W.reshape(L, d, d)

    hs = []
    for i in range(L):
        S, M, G = _sc_reduce(h, dstp, srcp, rsep, meta)
        h = _layer_dense(h, S, M, G, invc, rd, na_W[i], na_b[i],
                         na_w[i], l2W, lin2_b)
        hs.append(h)

    return _head(jnp.stack(hs), sc_w[:, 1], la_w[0], laW, la_b,
                 cls_W1, cls_b1, cls_W2, cls_b2)


# speedup vs baseline: 2.3391x; 1.0001x over previous
import functools

import jax
import jax.numpy as jnp
from jax import lax
from jax.experimental import pallas as pl
from jax.experimental.pallas import tpu as pltpu
from jax.experimental.pallas import tpu_sc as plsc


BLK = 1000
NW = 32
RPT = 2
NT = NW * RPT
NPT = 160
NPAD = NT * NPT
CHUNK = 128
NEG = -1.0e30


def _elu(v):
    return jnp.where(v > 0, v, jnp.exp(jnp.minimum(v, 0.0)) - 1.0)


def _sc_body(EPAD, htab, dstp, srcp, rsep, meta, S, M, G,
             outS, outMa, outMb, outG, stage, srcb, dstb, rseb, metab,
             sem0, sem1):
    cid = lax.axis_index("c")
    sid = lax.axis_index("s")
    wid = sid * 2 + cid
    sems = (sem0, sem1)

    def pass_body(it, _):
        br = it >> 1
        rr = it & 1
        if True:
            tile = wid * RPT + rr
            moff = pl.multiple_of((br * NT + tile) * 16, 16)
            pltpu.sync_copy(meta.at[pl.ds(moff, 16)], metab)
            mv = metab[...]
            e0 = pl.multiple_of(mv[0] + br * EPAD, 8)
            nc = mv[1]
            vlo = tile * NPT

            def zbody(r, _):
                z = jnp.zeros((16,), jnp.float32)
                neg = jnp.full((16,), NEG, jnp.float32)
                for j in range(8):
                    sl = pl.ds(j * 16, 16)
                    outS[r, sl] = z
                    outG[r, sl] = z
                    outMa[r, sl] = neg
                    outMb[r, sl] = neg
                return 0

            lax.fori_loop(0, NPT + 8, zbody, 0)

            def load_idx(chunk, b):
                off = pl.multiple_of(e0 + chunk * CHUNK, 8)
                pltpu.sync_copy(srcp.at[pl.ds(off, CHUNK)], srcb.at[b])
                pltpu.sync_copy(dstp.at[pl.ds(off, CHUNK)], dstb.at[b])
                pltpu.sync_copy(rsep.at[pl.ds(off, CHUNK)], rseb.at[b])

            def gather(b):
                return pltpu.make_async_copy(
                    htab.at[srcb.at[b]], stage.at[b], sems[b])

            load_idx(0, 0)
            gather(0).start()

            def outer(c2, _):
                for b in range(2):
                    chunk = c2 * 2 + b

                    @pl.when(chunk < nc)
                    def _():
                        @pl.when(chunk + 1 < nc)
                        def _():
                            load_idx(chunk + 1, 1 - b)
                            gather(1 - b).start()

                        gather(b).wait()

                        def gbody(g, _):
                            base = g * 16
                            dv = dstb[b, pl.ds(base, 16)]
                            wv = rseb[b, pl.ds(base, 16)]
                            okv = (dv >= vlo) & (dv < vlo + NPT)
                            slotv = jnp.where(okv, dv - vlo, NPT)
                            for k in range(16):
                                slot = slotv[k]
                                w = wv[k]
                                e = base + k
                                outMx = outMa if k % 2 == 0 else outMb
                                for j in range(8):
                                    sl = pl.ds(j * 16, 16)
                                    r = stage[b, e, sl]
                                    plsc.addupdate(outS.at[slot, sl], r)
                                    plsc.addupdate(outG.at[slot, sl], w * r)
                                    outMx[slot, sl] = jnp.maximum(
                                        outMx[slot, sl], r)
                            return 0

                        lax.fori_loop(0, CHUNK // 16, gbody, 0)
                return 0

            lax.fori_loop(0, (nc + 1) // 2, outer, 0)

            def mbody(r, _):
                for j in range(8):
                    sl = pl.ds(j * 16, 16)
                    outMa[r, sl] = jnp.maximum(outMa[r, sl], outMb[r, sl])
                return 0

            lax.fori_loop(0, NPT, mbody, 0)

            vo = pl.multiple_of(vlo, 8)
            pltpu.sync_copy(outS.at[pl.ds(0, NPT)], S.at[br, pl.ds(vo, NPT)])
            pltpu.sync_copy(outMa.at[pl.ds(0, NPT)], M.at[br, pl.ds(vo, NPT)])
            pltpu.sync_copy(outG.at[pl.ds(0, NPT)], G.at[br, pl.ds(vo, NPT)])
        return 0

    lax.fori_loop(0, 3 * RPT, pass_body, 0)


def _sc_reduce(h, dstp, srcp, rsep, meta):
    out = jax.ShapeDtypeStruct((3, NPAD, 128), jnp.float32)
    k = pl.kernel(
        functools.partial(_sc_body, dstp.shape[0] // 3),
        out_type=[out, out, out],
        mesh=plsc.VectorSubcoreMesh(core_axis_name="c", subcore_axis_name="s"),
        scratch_types=[
            pltpu.VMEM((NPT + 8, 128), jnp.float32),
            pltpu.VMEM((NPT + 8, 128), jnp.float32),
            pltpu.VMEM((NPT + 8, 128), jnp.float32),
            pltpu.VMEM((NPT + 8, 128), jnp.float32),
            pltpu.VMEM((2, CHUNK, 128), jnp.float32),
            pltpu.VMEM((2, CHUNK), jnp.int32),
            pltpu.VMEM((2, CHUNK), jnp.int32),
            pltpu.VMEM((2, CHUNK), jnp.float32),
            pltpu.VMEM((16,), jnp.int32),
            pltpu.SemaphoreType.DMA,
            pltpu.SemaphoreType.DMA,
        ],
    )
    return k(h, dstp, srcp, rsep, meta)


def _lin1_body(x_ref, w_ref, b_ref, o_ref):
    o_ref[...] = (
        jnp.dot(x_ref[...], w_ref[...], preferred_element_type=jnp.float32)
        + b_ref[...])


def _lin1(x, W, b):
    n, d = x.shape
    return pl.pallas_call(
        _lin1_body,
        grid=(n // BLK,),
        in_specs=[
            pl.BlockSpec((BLK, d), lambda i: (i, 0)),
            pl.BlockSpec((d, d), lambda i: (0, 0)),
            pl.BlockSpec((1, d), lambda i: (0, 0)),
        ],
        out_specs=pl.BlockSpec((BLK, d), lambda i: (i, 0)),
        out_shape=jax.ShapeDtypeStruct((n, d), jnp.float32),
    )(x, W, b.reshape(1, -1))


def _layer_body(h_ref, S_ref, M_ref, G_ref, invc_ref, rd_ref, naW_ref,
                nab_ref, naw_ref, l2W_ref, l2b_ref, o_ref):
    h = h_ref[...]
    acc = None
    for br in range(3):
        S = S_ref[br]
        Mr = M_ref[br]
        Mr = jnp.where(Mr > -1.0e29, Mr, 0.0)
        Gr = G_ref[br]
        aggs = (S * invc_ref[br], Mr, Gr * rd_ref[br], h + S)
        mixed = None
        for p in range(4):
            t = _elu(
                jnp.dot(aggs[p], naW_ref[br, p],
                        preferred_element_type=jnp.float32)
                + nab_ref[br, p]
            )
            t = naw_ref[0, p] * t
            mixed = t if mixed is None else mixed + t
        part = jnp.dot(mixed, l2W_ref[br], preferred_element_type=jnp.float32)
        acc = part if acc is None else acc + part
    o_ref[...] = jnp.maximum(acc + l2b_ref[...], 0.0)


def _layer_dense(h, S, M, G, invc, rd, naW, nab, naw, l2W, l2b):
    n, d = h.shape
    return pl.pallas_call(
        _layer_body,
        grid=(n // BLK,),
        in_specs=[
            pl.BlockSpec((BLK, d), lambda i: (i, 0)),
            pl.BlockSpec((3, BLK, d), lambda i: (0, i, 0)),
            pl.BlockSpec((3, BLK, d), lambda i: (0, i, 0)),
            pl.BlockSpec((3, BLK, d), lambda i: (0, i, 0)),
            pl.BlockSpec((3, BLK, 1), lambda i: (0, i, 0)),
            pl.BlockSpec((3, BLK, 1), lambda i: (0, i, 0)),
            pl.BlockSpec((3, 4, d, d), lambda i: (0, 0, 0, 0)),
            pl.BlockSpec((3, 4, 1, d), lambda i: (0, 0, 0, 0)),
            pl.BlockSpec((1, 4), lambda i: (0, 0)),
            pl.BlockSpec((3, d, d), lambda i: (0, 0, 0)),
            pl.BlockSpec((1, d), lambda i: (0, 0)),
        ],
        out_specs=pl.BlockSpec((BLK, d), lambda i: (i, 0)),
        out_shape=jax.ShapeDtypeStruct((n, d), jnp.float32),
    )(h, S, M, G, invc, rd, naW, nab.reshape(3, 4, 1, d), naw.reshape(1, 4),
      l2W, l2b.reshape(1, d))


def _head_body(hs_ref, scw_ref, law_ref, laW_ref, lab_ref,
               c1W_ref, c1b_ref, c2W_ref, c2b_ref, o_ref):
    jk = [scw_ref[0, i] * hs_ref[i] for i in range(3)]
    l_max = jnp.maximum(jnp.maximum(jk[0], jk[1]), jk[2])
    l_mean = (jk[0] + jk[1] + jk[2]) * (1.0 / 3.0)
    l_cat = None
    for i in range(3):
        t = jnp.dot(jk[i], laW_ref[i], preferred_element_type=jnp.float32)
        l_cat = t if l_cat is None else l_cat + t
    l_cat = l_cat + lab_ref[...]
    merge = (law_ref[0, 0] * jnp.maximum(l_max, 0.0)
             + law_ref[0, 1] * jnp.maximum(l_mean, 0.0)
             + law_ref[0, 2] * jnp.maximum(l_cat, 0.0))
    merge = jnp.maximum(merge, 0.0)
    hid = jnp.maximum(
        jnp.dot(merge, c1W_ref[...], preferred_element_type=jnp.float32)
        + c1b_ref[...], 0.0)
    o_ref[...] = (
        jnp.dot(hid, c2W_ref[...], preferred_element_type=jnp.float32)
        + c2b_ref[...])


def _head(hs, scw, law, laW, lab, c1W, c1b, c2W, c2b):
    _, n, d = hs.shape
    od = c2W.shape[1]
    return pl.pallas_call(
        _head_body,
        grid=(n // BLK,),
        in_specs=[
            pl.BlockSpec((3, BLK, d), lambda i: (0, i, 0)),
            pl.BlockSpec((1, 3), lambda i: (0, 0)),
            pl.BlockSpec((1, 3), lambda i: (0, 0)),
            pl.BlockSpec((3, d, d), lambda i: (0, 0, 0)),
            pl.BlockSpec((1, d), lambda i: (0, 0)),
            pl.BlockSpec((d, d), lambda i: (0, 0)),
            pl.BlockSpec((1, d), lambda i: (0, 0)),
            pl.BlockSpec((d, od), lambda i: (0, 0)),
            pl.BlockSpec((1, od), lambda i: (0, 0)),
        ],
        out_specs=pl.BlockSpec((BLK, od), lambda i: (i, 0)),
        out_shape=jax.ShapeDtypeStruct((n, od), jnp.float32),
    )(hs, scw.reshape(1, 3), law.reshape(1, 3), laW, lab.reshape(1, d),
      c1W, c1b.reshape(1, d), c2W, c2b.reshape(1, od))


def kernel(x, edge_index, na_alphas, sc_alphas, la_alphas, lin1_W, lin1_b,
           lin2_W, lin2_b, na_W, na_b, la_W, la_b, cls_W1, cls_b1,
           cls_W2, cls_b2):
    n, d = x.shape
    L = na_W.shape[0]
    E = edge_index.shape[2]
    EP = E + 2 * CHUNK
    na_w = jax.nn.softmax(na_alphas, axis=-1)
    sc_w = jax.nn.softmax(sc_alphas, axis=-1)
    la_w = jax.nn.softmax(la_alphas, axis=-1)

    dstps, srcps, rseps, metas, invcs, rds = [], [], [], [], [], []
    vb = jnp.minimum(jnp.arange(NT + 1, dtype=jnp.int32) * NPT, n)
    for br in range(3):
        src = edge_index[br, 0]
        dst = edge_index[br, 1]
        dst_s, src_s = lax.sort((dst, src), num_keys=1)
        ptr = jnp.searchsorted(dst_s, jnp.arange(n + 1, dtype=jnp.int32),
                               side="left").astype(jnp.int32)
        cnt = (ptr[1:] - ptr[:-1]).astype(jnp.float32)
        deg = jnp.maximum(cnt, 1.0)
        rs = lax.rsqrt(deg)
        invcs.append(1.0 / deg)
        rds.append(rs)
        rse = rs[src_s]
        e_b = ptr[vb]
        e_lo = e_b[:NT] & ~jnp.int32(7)
        nc = jnp.maximum((e_b[1:] - e_lo + (CHUNK - 1)) // CHUNK, 1)
        metas.append(jnp.stack(
            [e_lo, nc, jnp.zeros_like(e_lo), jnp.zeros_like(e_lo)], axis=1))
        dstps.append(jnp.concatenate(
            [dst_s, jnp.full((EP - E,), NPAD, jnp.int32)]))
        srcps.append(jnp.concatenate([src_s, jnp.zeros((EP - E,), jnp.int32)]))
        rseps.append(jnp.concatenate([rse, jnp.zeros((EP - E,), jnp.float32)]))
    dstp = jnp.concatenate(dstps)
    srcp = jnp.concatenate(srcps)
    rsep = jnp.concatenate(rseps)
    meta = jnp.pad(jnp.stack(metas), ((0, 0), (0, 0), (0, 12))).reshape(-1)
    invc = jnp.stack(invcs)[..., None]
    rd = jnp.stack(rds)[..., None]

    h = _lin1(x, lin1_W, lin1_b)
    l2W = lin2_W.reshape(3, d, d)
    laW = la_W.reshape(L, d, d)

    hs = []
    for i in range(L):
        S, M, G = _sc_reduce(h, dstp, srcp, rsep, meta)
        h = _layer_dense(h, S, M, G, invc, rd, na_W[i], na_b[i],
                         na_w[i], l2W, lin2_b)
        hs.append(h)

    return _head(jnp.stack(hs), sc_w[:, 1], la_w[0], laW, la_b,
                 cls_W1, cls_b1, cls_W2, cls_b2)
